# baseline (speedup 1.0000x reference)
import functools

import jax
import jax.numpy as jnp
from jax import lax
from jax.experimental import pallas as pl
from jax.experimental.pallas import tpu as pltpu
from jax.experimental.pallas import tpu_sc as plsc

N_NODES = 10000
N_EDGES = 320000
D_NODE = 128
D_EDGE = 16
H = 128
L = 4
G = 64

NC = 2
NS = 16
NW = NC * NS
EPW = N_EDGES // NW
EB_SC = 40
CHUNKS_PER_W = EPW // EB_SC
ROWS_PER_SUB = 624
ROWS_TAIL = N_NODES - NS * ROWS_PER_SUB

NODE_BLK = 2000
EDGE_BLK = 8000


def _embed_body(x_ref, w1_ref, b1_ref, w2_ref, b2_ref, o_ref):
    t = jnp.maximum(
        jnp.dot(x_ref[...], w1_ref[...], preferred_element_type=jnp.float32)
        + b1_ref[...], 0.0)
    o_ref[...] = (
        jnp.dot(t, w2_ref[...], preferred_element_type=jnp.float32)
        + b2_ref[...])


def _embed(x, w1, b1, w2, b2):
    nblk = N_NODES // NODE_BLK
    return pl.pallas_call(
        _embed_body,
        grid=(nblk,),
        in_specs=[
            pl.BlockSpec((NODE_BLK, D_NODE), lambda i: (i, 0)),
            pl.BlockSpec((D_NODE, H), lambda i: (0, 0)),
            pl.BlockSpec((1, H), lambda i: (0, 0)),
            pl.BlockSpec((H, H), lambda i: (0, 0)),
            pl.BlockSpec((1, H), lambda i: (0, 0)),
        ],
        out_specs=pl.BlockSpec((NODE_BLK, H), lambda i: (i, 0)),
        out_shape=jax.ShapeDtypeStruct((N_NODES, H), jnp.float32),
    )(x, w1, b1, w2, b2)


def _eall_body(ea_ref, we_ref, be_ref, o_ref):
    o_ref[...] = (
        jnp.dot(ea_ref[...], we_ref[...], preferred_element_type=jnp.float32)
        + be_ref[...])


def _edge_features(edge_attr, We_l, be_l):
    neb = N_EDGES // EDGE_BLK
    return pl.pallas_call(
        _eall_body,
        grid=(neb,),
        in_specs=[
            pl.BlockSpec((EDGE_BLK, D_EDGE), lambda j: (j, 0)),
            pl.BlockSpec((D_EDGE, H), lambda j: (0, 0)),
            pl.BlockSpec((1, H), lambda j: (0, 0)),
        ],
        out_specs=pl.BlockSpec((EDGE_BLK, H), lambda j: (j, 0)),
        out_shape=jax.ShapeDtypeStruct((N_EDGES, H), jnp.float32),
    )(edge_attr, We_l, be_l.reshape(1, H))


def _edge_features_all(edge_attr, We, be):
    neb = N_EDGES // EDGE_BLK
    return pl.pallas_call(
        lambda ea_ref, we_ref, be_ref, o_ref: o_ref.__setitem__(
            ..., jnp.dot(ea_ref[...], we_ref[0],
                         preferred_element_type=jnp.float32) + be_ref[0]),
        grid=(L, neb),
        in_specs=[
            pl.BlockSpec((EDGE_BLK, D_EDGE), lambda l, j: (j, 0)),
            pl.BlockSpec((1, D_EDGE, H), lambda l, j: (l, 0, 0)),
            pl.BlockSpec((1, 1, H), lambda l, j: (l, 0, 0)),
        ],
        out_specs=pl.BlockSpec((EDGE_BLK, H), lambda l, j: (l * neb + j, 0)),
        out_shape=jax.ShapeDtypeStruct((L * N_EDGES, H), jnp.float32),
    )(edge_attr, We, be.reshape(L, 1, H))


def _edge_pass_body(l, h_hbm, e_hbm, src_hbm, dst_hbm, out_hbm,
                    sb, db, rows_b, e_b, agg_sh,
                    sem_g0, sem_g1, sem_e0, sem_e1, sem_i):
    c = lax.axis_index("c")
    s = lax.axis_index("s")
    wid = c * NS + s
    row0 = pl.multiple_of(s * ROWS_PER_SUB, 8)
    pltpu.sync_copy(h_hbm.at[pl.ds(row0, ROWS_PER_SUB)],
                    agg_sh.at[pl.ds(row0, ROWS_PER_SUB)])

    @pl.when(s == 0)
    def _init_tail():
        pltpu.sync_copy(h_hbm.at[pl.ds(NS * ROWS_PER_SUB, ROWS_TAIL)],
                        agg_sh.at[pl.ds(NS * ROWS_PER_SUB, ROWS_TAIL)])

    plsc.subcore_barrier()

    ebase = l * N_EDGES + wid * EPW
    sem_g = (sem_g0, sem_g1)
    sem_e = (sem_e0, sem_e1)

    def issue_rows(ci, isl, slot):
        pltpu.async_copy(h_hbm.at[sb.at[isl]], rows_b.at[slot],
                         sem_g[slot])
        pltpu.async_copy(
            e_hbm.at[pl.ds(pl.multiple_of(ebase + ci * EB_SC, 8), EB_SC)],
            e_b.at[slot], sem_e[slot])

    def issue_idx(ci, isl):
        pltpu.async_copy(src_hbm.at[wid, ci], sb.at[isl], sem_i)
        pltpu.async_copy(dst_hbm.at[wid, ci], db.at[isl], sem_i)

    def wait_idx(isl):
        pltpu.make_async_copy(src_hbm.at[wid, 0], sb.at[isl], sem_i).wait()
        pltpu.make_async_copy(dst_hbm.at[wid, 0], db.at[isl], sem_i).wait()

    pltpu.sync_copy(src_hbm.at[wid, 0], sb.at[0])
    pltpu.sync_copy(dst_hbm.at[wid, 0], db.at[0])
    issue_rows(0, 0, 0)
    issue_idx(1, 1)

    def step(c0, j):
        ci = c0 + j
        rs = j % 2
        ns = (j + 1) % 2
        isl = j
        nisl = (j + 1) % 4
        n2isl = (j + 2) % 4

        @pl.when(ci + 1 < CHUNKS_PER_W)
        def _next_rows():
            wait_idx(nisl)
            issue_rows(ci + 1, nisl, ns)

        @pl.when(ci + 2 < CHUNKS_PER_W)
        def _next_idx():
            issue_idx(ci + 2, n2isl)

        @pl.when(ci < CHUNKS_PER_W)
        def _consume():
            pltpu.make_async_copy(h_hbm.at[sb.at[isl]], rows_b.at[rs],
                                  sem_g[rs]).wait()
            pltpu.make_async_copy(e_hbm.at[pl.ds(0, EB_SC)], e_b.at[rs],
                                  sem_e[rs]).wait()

            def _row(i2, carry):
                for r in range(2):
                    i = 2 * i2 + r
                    for jj in range(H // 16):
                        sl = pl.ds(jj * 16, 16)
                        rows_b[rs, i, sl] = jnp.maximum(
                            rows_b[rs, i, sl] + e_b[rs, i, sl], 0.0)
                return carry

            lax.fori_loop(0, EB_SC // 2, _row, 0)

            pltpu.sync_copy(rows_b.at[rs], agg_sh.at[db.at[isl]], add=True)

    def quad(k, carry):
        c0 = pl.multiple_of(4 * k, 4)
        for j in range(4):
            step(c0, j)
        return carry

    lax.fori_loop(0, (CHUNKS_PER_W + 3) // 4, quad, 0)
    plsc.subcore_barrier()
    pltpu.sync_copy(agg_sh.at[pl.ds(row0, ROWS_PER_SUB)],
                    out_hbm.at[c, pl.ds(row0, ROWS_PER_SUB)])

    @pl.when(s == 0)
    def _out_tail():
        pltpu.sync_copy(agg_sh.at[pl.ds(NS * ROWS_PER_SUB, ROWS_TAIL)],
                        out_hbm.at[c, pl.ds(NS * ROWS_PER_SUB, ROWS_TAIL)])


def _edge_pass(l, h, e_all, src3, dst3):
    mesh = plsc.VectorSubcoreMesh(core_axis_name="c", subcore_axis_name="s",
                                  num_cores=NC, num_subcores=NS)
    f = functools.partial(
        pl.kernel,
        out_type=jax.ShapeDtypeStruct((NC, N_NODES, H), jnp.float32),
        mesh=mesh,
        scratch_types=[
            pltpu.VMEM((4, EB_SC), jnp.int32),
            pltpu.VMEM((4, EB_SC), jnp.int32),
            pltpu.VMEM((2, EB_SC, H), jnp.float32),
            pltpu.VMEM((2, EB_SC, H), jnp.float32),
            pltpu.VMEM_SHARED((N_NODES, H), jnp.float32),
            pltpu.SemaphoreType.DMA,
            pltpu.SemaphoreType.DMA,
            pltpu.SemaphoreType.DMA,
            pltpu.SemaphoreType.DMA,
            pltpu.SemaphoreType.DMA,
        ],
    )(functools.partial(_edge_pass_body, l))
    return f(h, e_all, src3, dst3)


def _mlp_body(h_ref, p_ref, w1_ref, b1_ref, w2_ref, b2_ref, g_ref, bb_ref,
              o_ref):
    h = h_ref[...]
    zin = p_ref[0] + p_ref[1] - h
    z = jnp.maximum(
        jnp.dot(zin, w1_ref[...], preferred_element_type=jnp.float32)
        + b1_ref[...], 0.0)
    z = (jnp.dot(z, w2_ref[...], preferred_element_type=jnp.float32)
         + b2_ref[...])
    mu = jnp.mean(z, axis=-1, keepdims=True)
    zc = z - mu
    var = jnp.mean(zc * zc, axis=-1, keepdims=True)
    z = zc * lax.rsqrt(var + 1e-5) * g_ref[...] + bb_ref[...]
    o_ref[...] = h + jnp.maximum(z, 0.0)


def _mlp_update(h, p, w1, b1, w2, b2, g, b):
    nblk = N_NODES // NODE_BLK
    return pl.pallas_call(
        _mlp_body,
        grid=(nblk,),
        in_specs=[
            pl.BlockSpec((NODE_BLK, H), lambda i: (i, 0)),
            pl.BlockSpec((NC, NODE_BLK, H), lambda i: (0, i, 0)),
            pl.BlockSpec((H, 2 * H), lambda i: (0, 0)),
            pl.BlockSpec((1, 2 * H), lambda i: (0, 0)),
            pl.BlockSpec((2 * H, H), lambda i: (0, 0)),
            pl.BlockSpec((1, H), lambda i: (0, 0)),
            pl.BlockSpec((1, H), lambda i: (0, 0)),
            pl.BlockSpec((1, H), lambda i: (0, 0)),
        ],
        out_specs=pl.BlockSpec((NODE_BLK, H), lambda i: (i, 0)),
        out_shape=jax.ShapeDtypeStruct((N_NODES, H), jnp.float32),
    )(h, p, w1, b1, w2, b2, g, b)


def _pool_body(h_ref, b_ref, w1a_ref, w1b_ref, br1_ref, w2_ref, br2_ref,
               w3_ref, br3_ref, o_ref, sums, cnts, maxs):
    i = pl.program_id(0)

    @pl.when(i == 0)
    def _init():
        sums[...] = jnp.zeros_like(sums)
        cnts[...] = jnp.zeros_like(cnts)
        maxs[...] = jnp.full_like(maxs, -jnp.inf)

    h = h_ref[...]
    b = b_ref[...]
    gid = lax.broadcasted_iota(jnp.int32, (NODE_BLK, G), 1)
    onehot = (b == gid).astype(jnp.float32)
    sums[...] += lax.dot_general(onehot, h, (((0,), (0,)), ((), ())),
                                 preferred_element_type=jnp.float32)
    cnts[...] += lax.dot_general(onehot, jnp.ones_like(h),
                                 (((0,), (0,)), ((), ())),
                                 preferred_element_type=jnp.float32)
    for g in range(G):
        mval = jnp.max(jnp.where(b == g, h, -jnp.inf), axis=0, keepdims=True)
        maxs[g:g + 1, :] = jnp.maximum(maxs[g:g + 1, :], mval)

    @pl.when(i == pl.num_programs(0) - 1)
    def _fin():
        mean = sums[...] / jnp.maximum(cnts[...], 1.0)
        o1 = jnp.maximum(
            jnp.dot(mean, w1a_ref[...], preferred_element_type=jnp.float32)
            + jnp.dot(maxs[...], w1b_ref[...],
                      preferred_element_type=jnp.float32)
            + br1_ref[...], 0.0)
        o2 = jnp.maximum(
            jnp.dot(o1, w2_ref[...], preferred_element_type=jnp.float32)
            + br2_ref[...], 0.0)
        o_ref[...] = (
            jnp.dot(o2, w3_ref[...], preferred_element_type=jnp.float32)
            + br3_ref[...])


def _pool_readout(h, batch2d, w1a, w1b, br1, w2, br2, w3, br3):
    nblk = N_NODES // NODE_BLK
    return pl.pallas_call(
        _pool_body,
        grid=(nblk,),
        in_specs=[
            pl.BlockSpec((NODE_BLK, H), lambda i: (i, 0)),
            pl.BlockSpec((NODE_BLK, 1), lambda i: (i, 0)),
            pl.BlockSpec((H, H), lambda i: (0, 0)),
            pl.BlockSpec((H, H), lambda i: (0, 0)),
            pl.BlockSpec((1, H), lambda i: (0, 0)),
            pl.BlockSpec((H, H // 2), lambda i: (0, 0)),
            pl.BlockSpec((1, H // 2), lambda i: (0, 0)),
            pl.BlockSpec((H // 2, 1), lambda i: (0, 0)),
            pl.BlockSpec((1, 1), lambda i: (0, 0)),
        ],
        out_specs=pl.BlockSpec((G, 1), lambda i: (0, 0)),
        out_shape=jax.ShapeDtypeStruct((G, 1), jnp.float32),
        scratch_shapes=[
            pltpu.VMEM((G, H), jnp.float32),
            pltpu.VMEM((G, H), jnp.float32),
            pltpu.VMEM((G, H), jnp.float32),
        ],
    )(h, batch2d, w1a, w1b, br1, w2, br2, w3, br3)


def kernel(x, edge_index, edge_attr, batch, W_emb1, b_emb1, W_emb2, b_emb2,
           We, be, Wm1, bm1, bn_g, bn_b, Wm2, bm2, ln_g, ln_b,
           Wr1, br1, Wr2, br2, Wr3, br3):
    src3 = edge_index[0].reshape(NW, CHUNKS_PER_W, EB_SC)
    dst3 = edge_index[1].reshape(NW, CHUNKS_PER_W, EB_SC)

    bn_s = bn_g * (1.0 / jnp.sqrt(1.0 + 1e-5))
    Wm1f = Wm1 * bn_s[:, None, :]
    bm1f = bm1 * bn_s + bn_b

    h = _embed(x, W_emb1, b_emb1.reshape(1, H), W_emb2, b_emb2.reshape(1, H))
    e_all = _edge_features_all(edge_attr, We, be)

    for l in range(L):
        p = _edge_pass(l, h, e_all, src3, dst3)
        h = _mlp_update(h, p, Wm1f[l], bm1f[l].reshape(1, 2 * H),
                        Wm2[l], bm2[l].reshape(1, H),
                        ln_g[l].reshape(1, H), ln_b[l].reshape(1, H))

    return _pool_readout(h, batch.reshape(N_NODES, 1),
                         Wr1[:H], Wr1[H:], br1.reshape(1, H),
                         Wr2, br2.reshape(1, H // 2),
                         Wr3, br3.reshape(1, 1))


# speedup vs baseline: 1.0466x; 1.0466x over previous
import functools

import jax
import jax.numpy as jnp
from jax import lax
from jax.experimental import pallas as pl
from jax.experimental.pallas import tpu as pltpu
from jax.experimental.pallas import tpu_sc as plsc

N_NODES = 10000
N_EDGES = 320000
D_NODE = 128
D_EDGE = 16
H = 128
L = 4
G = 64

NC = 2
NS = 16
NW = NC * NS
EPW = N_EDGES // NW
EB_SC = 40
CHUNKS_PER_W = EPW // EB_SC
ROWS_PER_SUB = 624
ROWS_TAIL = N_NODES - NS * ROWS_PER_SUB

NODE_BLK = 2000
EDGE_BLK = 8000


def _embed_body(x_ref, w1_ref, b1_ref, w2_ref, b2_ref, o_ref):
    t = jnp.maximum(
        jnp.dot(x_ref[...], w1_ref[...], preferred_element_type=jnp.float32)
        + b1_ref[...], 0.0)
    o_ref[...] = (
        jnp.dot(t, w2_ref[...], preferred_element_type=jnp.float32)
        + b2_ref[...])


def _embed(x, w1, b1, w2, b2):
    nblk = N_NODES // NODE_BLK
    return pl.pallas_call(
        _embed_body,
        grid=(nblk,),
        in_specs=[
            pl.BlockSpec((NODE_BLK, D_NODE), lambda i: (i, 0)),
            pl.BlockSpec((D_NODE, H), lambda i: (0, 0)),
            pl.BlockSpec((1, H), lambda i: (0, 0)),
            pl.BlockSpec((H, H), lambda i: (0, 0)),
            pl.BlockSpec((1, H), lambda i: (0, 0)),
        ],
        out_specs=pl.BlockSpec((NODE_BLK, H), lambda i: (i, 0)),
        out_shape=jax.ShapeDtypeStruct((N_NODES, H), jnp.float32),
    )(x, w1, b1, w2, b2)


def _eall_body(ea_ref, we_ref, be_ref, o_ref):
    o_ref[...] = (
        jnp.dot(ea_ref[...], we_ref[...], preferred_element_type=jnp.float32)
        + be_ref[...])


def _edge_features(edge_attr, We_l, be_l):
    neb = N_EDGES // EDGE_BLK
    return pl.pallas_call(
        _eall_body,
        grid=(neb,),
        in_specs=[
            pl.BlockSpec((EDGE_BLK, D_EDGE), lambda j: (j, 0)),
            pl.BlockSpec((D_EDGE, H), lambda j: (0, 0)),
            pl.BlockSpec((1, H), lambda j: (0, 0)),
        ],
        out_specs=pl.BlockSpec((EDGE_BLK, H), lambda j: (j, 0)),
        out_shape=jax.ShapeDtypeStruct((N_EDGES, H), jnp.float32),
    )(edge_attr, We_l, be_l.reshape(1, H))


def _edge_features_all(edge_attr, We, be):
    neb = N_EDGES // EDGE_BLK
    return pl.pallas_call(
        lambda ea_ref, we_ref, be_ref, o_ref: o_ref.__setitem__(
            ..., jnp.dot(ea_ref[...], we_ref[0],
                         preferred_element_type=jnp.float32) + be_ref[0]),
        grid=(L, neb),
        in_specs=[
            pl.BlockSpec((EDGE_BLK, D_EDGE), lambda l, j: (j, 0)),
            pl.BlockSpec((1, D_EDGE, H), lambda l, j: (l, 0, 0)),
            pl.BlockSpec((1, 1, H), lambda l, j: (l, 0, 0)),
        ],
        out_specs=pl.BlockSpec((EDGE_BLK, H), lambda l, j: (l * neb + j, 0)),
        out_shape=jax.ShapeDtypeStruct((L * N_EDGES, H), jnp.float32),
    )(edge_attr, We, be.reshape(L, 1, H))


def _edge_pass_body(l, h_hbm, e_hbm, src_hbm, dst_hbm, out_hbm,
                    sb, db, rows_b, e_b, agg_sh,
                    sem_g0, sem_g1, sem_g2, sem_e0, sem_e1, sem_i,
                    sem_s0, sem_s1, sem_s2):
    c = lax.axis_index("c")
    s = lax.axis_index("s")
    wid = c * NS + s
    row0 = pl.multiple_of(s * ROWS_PER_SUB, 8)
    pltpu.sync_copy(h_hbm.at[pl.ds(row0, ROWS_PER_SUB)],
                    agg_sh.at[pl.ds(row0, ROWS_PER_SUB)])

    @pl.when(s == 0)
    def _init_tail():
        pltpu.sync_copy(h_hbm.at[pl.ds(NS * ROWS_PER_SUB, ROWS_TAIL)],
                        agg_sh.at[pl.ds(NS * ROWS_PER_SUB, ROWS_TAIL)])

    plsc.subcore_barrier()

    ebase = l * N_EDGES + wid * EPW
    sem_g = (sem_g0, sem_g1, sem_g2)
    sem_e = (sem_e0, sem_e1)
    sem_s = (sem_s0, sem_s1, sem_s2)

    def issue_rows(ci, isl, slot, eslot):
        pltpu.async_copy(h_hbm.at[sb.at[isl]], rows_b.at[slot],
                         sem_g[slot])
        pltpu.async_copy(
            e_hbm.at[pl.ds(pl.multiple_of(ebase + ci * EB_SC, 8), EB_SC)],
            e_b.at[eslot], sem_e[eslot])

    def issue_idx(ci, isl):
        pltpu.async_copy(src_hbm.at[wid, ci], sb.at[isl], sem_i)
        pltpu.async_copy(dst_hbm.at[wid, ci], db.at[isl], sem_i)

    def wait_idx(isl):
        pltpu.make_async_copy(src_hbm.at[wid, 0], sb.at[isl], sem_i).wait()
        pltpu.make_async_copy(dst_hbm.at[wid, 0], db.at[isl], sem_i).wait()

    pltpu.sync_copy(src_hbm.at[wid, 0], sb.at[0])
    pltpu.sync_copy(dst_hbm.at[wid, 0], db.at[0])
    issue_rows(0, 0, 0, 0)
    issue_idx(1, 1)

    def drain_scatter(slot):
        pltpu.make_async_copy(rows_b.at[slot], agg_sh.at[db.at[0]],
                              sem_s[slot]).wait()

    def step(c0, j):
        ci = c0 + j
        rs = j % 3
        ns = (j + 1) % 3
        es = j % 2
        nes = (j + 1) % 2
        isl = j % 4
        nisl = (j + 1) % 4
        n2isl = (j + 2) % 4

        @pl.when(ci + 1 < CHUNKS_PER_W)
        def _next_rows():
            wait_idx(nisl)

            @pl.when(ci > 1)
            def _drain():
                drain_scatter(ns)

            issue_rows(ci + 1, nisl, ns, nes)

        @pl.when(ci + 2 < CHUNKS_PER_W)
        def _next_idx():
            issue_idx(ci + 2, n2isl)

        @pl.when(ci < CHUNKS_PER_W)
        def _consume():
            pltpu.make_async_copy(h_hbm.at[sb.at[isl]], rows_b.at[rs],
                                  sem_g[rs]).wait()
            pltpu.make_async_copy(e_hbm.at[pl.ds(0, EB_SC)], e_b.at[es],
                                  sem_e[es]).wait()

            def _row(i2, carry):
                for r in range(2):
                    i = 2 * i2 + r
                    for jj in range(H // 16):
                        sl = pl.ds(jj * 16, 16)
                        rows_b[rs, i, sl] = jnp.maximum(
                            rows_b[rs, i, sl] + e_b[es, i, sl], 0.0)
                return carry

            lax.fori_loop(0, EB_SC // 2, _row, 0)

            pltpu.async_copy(rows_b.at[rs], agg_sh.at[db.at[isl]],
                             sem_s[rs], add=True)

    def twelve(k, carry):
        c0 = pl.multiple_of(12 * k, 12)
        for j in range(12):
            step(c0, j)
        return carry

    lax.fori_loop(0, (CHUNKS_PER_W + 11) // 12, twelve, 0)
    drain_scatter((CHUNKS_PER_W - 3) % 3)
    drain_scatter((CHUNKS_PER_W - 2) % 3)
    drain_scatter((CHUNKS_PER_W - 1) % 3)
    plsc.subcore_barrier()
    pltpu.sync_copy(agg_sh.at[pl.ds(row0, ROWS_PER_SUB)],
                    out_hbm.at[c, pl.ds(row0, ROWS_PER_SUB)])

    @pl.when(s == 0)
    def _out_tail():
        pltpu.sync_copy(agg_sh.at[pl.ds(NS * ROWS_PER_SUB, ROWS_TAIL)],
                        out_hbm.at[c, pl.ds(NS * ROWS_PER_SUB, ROWS_TAIL)])


def _edge_pass(l, h, e_all, src3, dst3):
    mesh = plsc.VectorSubcoreMesh(core_axis_name="c", subcore_axis_name="s",
                                  num_cores=NC, num_subcores=NS)
    f = functools.partial(
        pl.kernel,
        out_type=jax.ShapeDtypeStruct((NC, N_NODES, H), jnp.float32),
        mesh=mesh,
        scratch_types=[
            pltpu.VMEM((4, EB_SC), jnp.int32),
            pltpu.VMEM((4, EB_SC), jnp.int32),
            pltpu.VMEM((3, EB_SC, H), jnp.float32),
            pltpu.VMEM((2, EB_SC, H), jnp.float32),
            pltpu.VMEM_SHARED((N_NODES, H), jnp.float32),
            pltpu.SemaphoreType.DMA,
            pltpu.SemaphoreType.DMA,
            pltpu.SemaphoreType.DMA,
            pltpu.SemaphoreType.DMA,
            pltpu.SemaphoreType.DMA,
            pltpu.SemaphoreType.DMA,
            pltpu.SemaphoreType.DMA,
            pltpu.SemaphoreType.DMA,
            pltpu.SemaphoreType.DMA,
        ],
    )(functools.partial(_edge_pass_body, l))
    return f(h, e_all, src3, dst3)


def _mlp_body(h_ref, p_ref, w1_ref, b1_ref, w2_ref, b2_ref, g_ref, bb_ref,
              o_ref):
    h = h_ref[...]
    zin = p_ref[0] + p_ref[1] - h
    z = jnp.maximum(
        jnp.dot(zin, w1_ref[...], preferred_element_type=jnp.float32)
        + b1_ref[...], 0.0)
    z = (jnp.dot(z, w2_ref[...], preferred_element_type=jnp.float32)
         + b2_ref[...])
    mu = jnp.mean(z, axis=-1, keepdims=True)
    zc = z - mu
    var = jnp.mean(zc * zc, axis=-1, keepdims=True)
    z = zc * lax.rsqrt(var + 1e-5) * g_ref[...] + bb_ref[...]
    o_ref[...] = h + jnp.maximum(z, 0.0)


def _mlp_update(h, p, w1, b1, w2, b2, g, b):
    nblk = N_NODES // NODE_BLK
    return pl.pallas_call(
        _mlp_body,
        grid=(nblk,),
        in_specs=[
            pl.BlockSpec((NODE_BLK, H), lambda i: (i, 0)),
            pl.BlockSpec((NC, NODE_BLK, H), lambda i: (0, i, 0)),
            pl.BlockSpec((H, 2 * H), lambda i: (0, 0)),
            pl.BlockSpec((1, 2 * H), lambda i: (0, 0)),
            pl.BlockSpec((2 * H, H), lambda i: (0, 0)),
            pl.BlockSpec((1, H), lambda i: (0, 0)),
            pl.BlockSpec((1, H), lambda i: (0, 0)),
            pl.BlockSpec((1, H), lambda i: (0, 0)),
        ],
        out_specs=pl.BlockSpec((NODE_BLK, H), lambda i: (i, 0)),
        out_shape=jax.ShapeDtypeStruct((N_NODES, H), jnp.float32),
    )(h, p, w1, b1, w2, b2, g, b)


def _pool_body(h_ref, b_ref, w1a_ref, w1b_ref, br1_ref, w2_ref, br2_ref,
               w3_ref, br3_ref, o_ref, sums, cnts, maxs):
    i = pl.program_id(0)

    @pl.when(i == 0)
    def _init():
        sums[...] = jnp.zeros_like(sums)
        cnts[...] = jnp.zeros_like(cnts)
        maxs[...] = jnp.full_like(maxs, -jnp.inf)

    h = h_ref[...]
    b = b_ref[...]
    gid = lax.broadcasted_iota(jnp.int32, (NODE_BLK, G), 1)
    onehot = (b == gid).astype(jnp.float32)
    sums[...] += lax.dot_general(onehot, h, (((0,), (0,)), ((), ())),
                                 preferred_element_type=jnp.float32)
    cnts[...] += lax.dot_general(onehot, jnp.ones_like(h),
                                 (((0,), (0,)), ((), ())),
                                 preferred_element_type=jnp.float32)
    for g in range(G):
        mval = jnp.max(jnp.where(b == g, h, -jnp.inf), axis=0, keepdims=True)
        maxs[g:g + 1, :] = jnp.maximum(maxs[g:g + 1, :], mval)

    @pl.when(i == pl.num_programs(0) - 1)
    def _fin():
        mean = sums[...] / jnp.maximum(cnts[...], 1.0)
        o1 = jnp.maximum(
            jnp.dot(mean, w1a_ref[...], preferred_element_type=jnp.float32)
            + jnp.dot(maxs[...], w1b_ref[...],
                      preferred_element_type=jnp.float32)
            + br1_ref[...], 0.0)
        o2 = jnp.maximum(
            jnp.dot(o1, w2_ref[...], preferred_element_type=jnp.float32)
            + br2_ref[...], 0.0)
        o_ref[...] = (
            jnp.dot(o2, w3_ref[...], preferred_element_type=jnp.float32)
            + br3_ref[...])


def _pool_readout(h, batch2d, w1a, w1b, br1, w2, br2, w3, br3):
    nblk = N_NODES // NODE_BLK
    return pl.pallas_call(
        _pool_body,
        grid=(nblk,),
        in_specs=[
            pl.BlockSpec((NODE_BLK, H), lambda i: (i, 0)),
            pl.BlockSpec((NODE_BLK, 1), lambda i: (i, 0)),
            pl.BlockSpec((H, H), lambda i: (0, 0)),
            pl.BlockSpec((H, H), lambda i: (0, 0)),
            pl.BlockSpec((1, H), lambda i: (0, 0)),
            pl.BlockSpec((H, H // 2), lambda i: (0, 0)),
            pl.BlockSpec((1, H // 2), lambda i: (0, 0)),
            pl.BlockSpec((H // 2, 1), lambda i: (0, 0)),
            pl.BlockSpec((1, 1), lambda i: (0, 0)),
        ],
        out_specs=pl.BlockSpec((G, 1), lambda i: (0, 0)),
        out_shape=jax.ShapeDtypeStruct((G, 1), jnp.float32),
        scratch_shapes=[
            pltpu.VMEM((G, H), jnp.float32),
            pltpu.VMEM((G, H), jnp.float32),
            pltpu.VMEM((G, H), jnp.float32),
        ],
    )(h, batch2d, w1a, w1b, br1, w2, br2, w3, br3)


def kernel(x, edge_index, edge_attr, batch, W_emb1, b_emb1, W_emb2, b_emb2,
           We, be, Wm1, bm1, bn_g, bn_b, Wm2, bm2, ln_g, ln_b,
           Wr1, br1, Wr2, br2, Wr3, br3):
    src3 = edge_index[0].reshape(NW, CHUNKS_PER_W, EB_SC)
    dst3 = edge_index[1].reshape(NW, CHUNKS_PER_W, EB_SC)

    bn_s = bn_g * (1.0 / jnp.sqrt(1.0 + 1e-5))
    Wm1f = Wm1 * bn_s[:, None, :]
    bm1f = bm1 * bn_s + bn_b

    h = _embed(x, W_emb1, b_emb1.reshape(1, H), W_emb2, b_emb2.reshape(1, H))
    e_all = _edge_features_all(edge_attr, We, be)

    for l in range(L):
        p = _edge_pass(l, h, e_all, src3, dst3)
        h = _mlp_update(h, p, Wm1f[l], bm1f[l].reshape(1, 2 * H),
                        Wm2[l], bm2[l].reshape(1, H),
                        ln_g[l].reshape(1, H), ln_b[l].reshape(1, H))

    return _pool_readout(h, batch.reshape(N_NODES, 1),
                         Wr1[:H], Wr1[H:], br1.reshape(1, H),
                         Wr2, br2.reshape(1, H // 2),
                         Wr3, br3.reshape(1, 1))
